# baseline (speedup 1.0000x reference)
import functools

import jax
import jax.numpy as jnp
from jax import lax
from jax.experimental import pallas as pl
from jax.experimental.pallas import tpu as pltpu
from jax.experimental.pallas import tpu_sc as plsc

D = 128
SNP_DOMAIN = 2
_NC = 2
_NS = 16
_NW = _NC * _NS
_T = 256
_LN_EPS = 1e-12
_N_PAD = 1024


def _rsqrt_newton(x):
    i = plsc.bitcast(x, jnp.int32)
    i = 0x5F3759DF - lax.shift_right_arithmetic(i, 1)
    y = plsc.bitcast(i, jnp.float32)
    for _ in range(4):
        y = y * (1.5 - 0.5 * x * y * y)
    return y


def _make_sc_kernel(BL, n_pos):
    per_worker = BL // _NW
    n_chunks = per_worker // _T
    mesh = plsc.VectorSubcoreMesh(core_axis_name="c", subcore_axis_name="s")

    @functools.partial(
        pl.kernel,
        mesh=mesh,
        compiler_params=pltpu.CompilerParams(needs_layout_passes=False),
        out_type=jax.ShapeDtypeStruct((BL, D), jnp.float32),
        scratch_types=[
            pltpu.VMEM((5 * _T,), jnp.int32),
            pltpu.VMEM((_T,), jnp.int32),
            pltpu.VMEM((_T,), jnp.int32),
            pltpu.VMEM((_T,), jnp.int32),
            pltpu.VMEM((_T, D), jnp.float32),
            pltpu.VMEM((_T, D), jnp.float32),
            pltpu.VMEM((D, 16), jnp.float32),
            pltpu.VMEM((D, 16), jnp.float32),
            pltpu.SemaphoreType.DMA,
        ],
    )
    def body(ids_hbm, w_merged_hbm, w_pv_hbm, w_pos_hbm, gb_hbm, bb_hbm,
             out_hbm,
             ids_v, cidx_v, pv_v, mpos_v,
             rows_m, rows_o, gb_v, bb_v, sem):
        wid = lax.axis_index("s") * _NC + lax.axis_index("c")
        wbase = wid * per_worker
        pltpu.sync_copy(gb_hbm, gb_v)
        pltpu.sync_copy(bb_hbm, bb_v)
        riota = lax.iota(jnp.int32, 16)

        def chunk_body(g, carry):
            base = wbase + g * _T
            row = wid * n_chunks + g
            pltpu.sync_copy(ids_hbm.at[row], ids_v)

            for j in range(_T // 16):
                sl = pl.ds(j * 16, 16)
                dom = ids_v[pl.ds(0 * _T + j * 16, 16)]
                snp = ids_v[pl.ds(1 * _T + j * 16, 16)]
                pt = ids_v[pl.ds(2 * _T + j * 16, 16)]
                pv = ids_v[pl.ds(3 * _T + j * 16, 16)]
                pos = ids_v[pl.ds(4 * _T + j * 16, 16)]
                cidx_v[sl] = dom * 1600 + snp * 100 + pt
                pv_v[sl] = pv
                mpos_v[sl] = jnp.where(dom == SNP_DOMAIN, pos,
                                       n_pos + (pos & (_N_PAD - 1)))

            c1 = pltpu.async_copy(w_merged_hbm.at[cidx_v], rows_m, sem)
            c1.wait()
            c2 = pltpu.async_copy(w_pv_hbm.at[pv_v], rows_m, sem, add=True)
            c3 = pltpu.async_copy(w_pos_hbm.at[mpos_v], rows_m, sem, add=True)
            c2.wait()
            c3.wait()

            def grp_body(q, c_):
                ridx = riota + q * 16
                acc = jnp.zeros((16,), jnp.float32)
                acc2 = jnp.zeros((16,), jnp.float32)
                for c in range(D):
                    cv = jnp.full((16,), c, jnp.int32)
                    v = plsc.load_gather(rows_m, [ridx, cv])
                    acc = acc + v
                    acc2 = acc2 + v * v
                mean = acc * (1.0 / D)
                var = acc2 * (1.0 / D) - mean * mean
                rstd = _rsqrt_newton(var + _LN_EPS)
                for c in range(D):
                    cv = jnp.full((16,), c, jnp.int32)
                    v = plsc.load_gather(rows_m, [ridx, cv])
                    o = (v - mean) * rstd * gb_v[c, :] + bb_v[c, :]
                    plsc.store_scatter(rows_o, [ridx, cv], o)
                return c_

            lax.fori_loop(0, _T // 16, grp_body, 0)
            pltpu.sync_copy(rows_o, out_hbm.at[pl.ds(base, _T)])
            return carry

        lax.fori_loop(0, n_chunks, chunk_body, 0)

    return body


def kernel(domain_ids, snp_value_ids, snp_position_ids, phenotype_value_ids,
           phenotype_type_ids, is_padding, W_domain, W_snp, W_phen_val,
           W_phen_type, W_pos, ln_gamma, ln_beta):
    B, L = domain_ids.shape
    BL = B * L
    n_pos = W_pos.shape[0]
    per_worker = BL // _NW
    n_chunks = per_worker // _T
    w_merged = (W_domain[:, None, None, :] + W_snp[None, :, None, :]
                + W_phen_type[None, None, :, :]).reshape(-1, D)
    w_pos_ext = jnp.concatenate(
        [W_pos, jnp.zeros((_N_PAD, D), W_pos.dtype)], axis=0)
    ids = jnp.stack([
        domain_ids.reshape(-1), snp_value_ids.reshape(-1),
        phenotype_type_ids.reshape(-1), phenotype_value_ids.reshape(-1),
        snp_position_ids.reshape(-1)
    ]).astype(jnp.int32)
    ids = ids.reshape(5, _NW, n_chunks, _T).transpose(1, 2, 0, 3)
    ids = ids.reshape(_NW * n_chunks, 5 * _T)
    gb = jnp.broadcast_to(ln_gamma[:, None], (D, 16))
    bb = jnp.broadcast_to(ln_beta[:, None], (D, 16))

    sc = _make_sc_kernel(BL, n_pos)
    out = sc(ids, w_merged, W_phen_val, w_pos_ext, gb, bb)
    return out.reshape(B, L, D)


# speedup vs baseline: 2.3146x; 2.3146x over previous
import functools

import jax
import jax.numpy as jnp
from jax import lax
from jax.experimental import pallas as pl
from jax.experimental.pallas import tpu as pltpu
from jax.experimental.pallas import tpu_sc as plsc

D = 128
SNP_DOMAIN = 2
_NC = 2
_NS = 16
_NW = _NC * _NS
_T = 256
_LN_EPS = 1e-12
_N_PAD = 1024


def _rsqrt_newton(x):
    i = plsc.bitcast(x, jnp.int32)
    i = 0x5F3759DF - lax.shift_right_arithmetic(i, 1)
    y = plsc.bitcast(i, jnp.float32)
    for _ in range(4):
        y = y * (1.5 - 0.5 * x * y * y)
    return y


def _make_sc_kernel(BL, n_pos):
    per_worker = BL // _NW
    n_chunks = per_worker // _T
    mesh = plsc.VectorSubcoreMesh(core_axis_name="c", subcore_axis_name="s")

    @functools.partial(
        pl.kernel,
        mesh=mesh,
        compiler_params=pltpu.CompilerParams(needs_layout_passes=False),
        out_type=jax.ShapeDtypeStruct((BL, D), jnp.float32),
        scratch_types=[
            pltpu.VMEM((5 * _T,), jnp.int32),
            pltpu.VMEM((_T,), jnp.int32),
            pltpu.VMEM((_T,), jnp.int32),
            pltpu.VMEM((_T,), jnp.int32),
            pltpu.VMEM((_T, D), jnp.float32),
            pltpu.VMEM((_T, D), jnp.float32),
            pltpu.VMEM((16, 17), jnp.float32),
            pltpu.VMEM((16, 17), jnp.float32),
            pltpu.VMEM((D,), jnp.float32),
            pltpu.VMEM((D,), jnp.float32),
            pltpu.SemaphoreType.DMA,
        ],
    )
    def body(ids_hbm, w_merged_hbm, w_pv_hbm, w_pos_hbm, gamma_hbm, beta_hbm,
             out_hbm,
             ids_v, cidx_v, pv_v, mpos_v,
             rows_m, rows_o, tb_s, tb_q, gamma_v, beta_v, sem):
        wid = lax.axis_index("s") * _NC + lax.axis_index("c")
        wbase = wid * per_worker
        pltpu.sync_copy(gamma_hbm, gamma_v)
        pltpu.sync_copy(beta_hbm, beta_v)
        riota = lax.iota(jnp.int32, 16)

        def chunk_body(g, carry):
            base = wbase + g * _T
            row = wid * n_chunks + g
            pltpu.sync_copy(ids_hbm.at[row], ids_v)

            for j in range(_T // 16):
                sl = pl.ds(j * 16, 16)
                dom = ids_v[pl.ds(0 * _T + j * 16, 16)]
                snp = ids_v[pl.ds(1 * _T + j * 16, 16)]
                pt = ids_v[pl.ds(2 * _T + j * 16, 16)]
                pv = ids_v[pl.ds(3 * _T + j * 16, 16)]
                pos = ids_v[pl.ds(4 * _T + j * 16, 16)]
                cidx_v[sl] = dom * 1600 + snp * 100 + pt
                pv_v[sl] = pv
                mpos_v[sl] = jnp.where(dom == SNP_DOMAIN, pos,
                                       n_pos + (pos & (_N_PAD - 1)))

            c1 = pltpu.async_copy(w_merged_hbm.at[cidx_v], rows_m, sem)
            c1.wait()
            c2 = pltpu.async_copy(w_pv_hbm.at[pv_v], rows_m, sem, add=True)
            c3 = pltpu.async_copy(w_pos_hbm.at[mpos_v], rows_m, sem, add=True)
            c2.wait()
            c3.wait()

            def grp_body(q, c_):
                t0 = q * 16
                for i in range(16):
                    t = t0 + i
                    sv = rows_m[t, pl.ds(0, 16)]
                    qv = sv * sv
                    for k in range(1, D // 16):
                        v = rows_m[t, pl.ds(k * 16, 16)]
                        sv = sv + v
                        qv = qv + v * v
                    tb_s[i, pl.ds(0, 16)] = sv
                    tb_q[i, pl.ds(0, 16)] = qv
                tot = plsc.load_gather(tb_s, [riota, jnp.full((16,), 0,
                                                             jnp.int32)])
                tot2 = plsc.load_gather(tb_q, [riota, jnp.full((16,), 0,
                                                               jnp.int32)])
                for j in range(1, 16):
                    cj = jnp.full((16,), j, jnp.int32)
                    tot = tot + plsc.load_gather(tb_s, [riota, cj])
                    tot2 = tot2 + plsc.load_gather(tb_q, [riota, cj])
                mean = tot * (1.0 / D)
                var = tot2 * (1.0 / D) - mean * mean
                rstd = _rsqrt_newton(var + _LN_EPS)
                nbias = -mean * rstd
                for i in range(16):
                    t = t0 + i
                    sc_i = jnp.broadcast_to(rstd[i], (16,))
                    nb_i = jnp.broadcast_to(nbias[i], (16,))
                    for k in range(D // 16):
                        sl = pl.ds(k * 16, 16)
                        v = rows_m[t, sl]
                        rows_o[t, sl] = ((v * sc_i + nb_i) * gamma_v[sl]
                                         + beta_v[sl])
                return c_

            lax.fori_loop(0, _T // 16, grp_body, 0)
            pltpu.sync_copy(rows_o, out_hbm.at[pl.ds(base, _T)])
            return carry

        lax.fori_loop(0, n_chunks, chunk_body, 0)

    return body


def kernel(domain_ids, snp_value_ids, snp_position_ids, phenotype_value_ids,
           phenotype_type_ids, is_padding, W_domain, W_snp, W_phen_val,
           W_phen_type, W_pos, ln_gamma, ln_beta):
    B, L = domain_ids.shape
    BL = B * L
    n_pos = W_pos.shape[0]
    per_worker = BL // _NW
    n_chunks = per_worker // _T
    w_merged = (W_domain[:, None, None, :] + W_snp[None, :, None, :]
                + W_phen_type[None, None, :, :]).reshape(-1, D)
    w_pos_ext = jnp.concatenate(
        [W_pos, jnp.zeros((_N_PAD, D), W_pos.dtype)], axis=0)
    ids = jnp.stack([
        domain_ids.reshape(-1), snp_value_ids.reshape(-1),
        phenotype_type_ids.reshape(-1), phenotype_value_ids.reshape(-1),
        snp_position_ids.reshape(-1)
    ]).astype(jnp.int32)
    ids = ids.reshape(5, _NW, n_chunks, _T).transpose(1, 2, 0, 3)
    ids = ids.reshape(_NW * n_chunks, 5 * _T)
    sc = _make_sc_kernel(BL, n_pos)
    out = sc(ids, w_merged, W_phen_val, w_pos_ext, ln_gamma, ln_beta)
    return out.reshape(B, L, D)


# speedup vs baseline: 2.4880x; 1.0749x over previous
import functools

import jax
import jax.numpy as jnp
from jax import lax
from jax.experimental import pallas as pl
from jax.experimental.pallas import tpu as pltpu
from jax.experimental.pallas import tpu_sc as plsc

D = 128
SNP_DOMAIN = 2
_NC = 2
_NS = 16
_NW = _NC * _NS
_T = 128
_LN_EPS = 1e-12
_N_PAD = 1024


def _rsqrt_newton(x):
    i = plsc.bitcast(x, jnp.int32)
    i = 0x5F3759DF - lax.shift_right_arithmetic(i, 1)
    y = plsc.bitcast(i, jnp.float32)
    for _ in range(4):
        y = y * (1.5 - 0.5 * x * y * y)
    return y


def _make_sc_kernel(BL, n_pos):
    per_worker = BL // _NW
    n_chunks = per_worker // _T
    mesh = plsc.VectorSubcoreMesh(core_axis_name="c", subcore_axis_name="s")

    @functools.partial(
        pl.kernel,
        mesh=mesh,
        compiler_params=pltpu.CompilerParams(needs_layout_passes=False),
        out_type=jax.ShapeDtypeStruct((BL, D), jnp.float32),
        scratch_types=[
            pltpu.VMEM((5 * _T,), jnp.int32),
            pltpu.VMEM((_T,), jnp.int32),
            pltpu.VMEM((_T,), jnp.int32),
            pltpu.VMEM((_T,), jnp.int32),
            pltpu.VMEM((_T, D), jnp.float32),
            pltpu.VMEM((_T, D), jnp.float32),
            pltpu.VMEM((_T // 16, 16, 17), jnp.float32),
            pltpu.VMEM((_T // 16, 16, 17), jnp.float32),
            pltpu.VMEM((D,), jnp.float32),
            pltpu.VMEM((D,), jnp.float32),
            pltpu.SemaphoreType.DMA,
        ],
    )
    def body(ids_hbm, w_merged_hbm, w_pv_hbm, w_pos_hbm, gamma_hbm, beta_hbm,
             out_hbm,
             ids_v, cidx_v, pv_v, mpos_v,
             rows_m, rows_o, tb_s, tb_q, gamma_v, beta_v, sem):
        wid = lax.axis_index("s") * _NC + lax.axis_index("c")
        wbase = wid * per_worker
        pltpu.sync_copy(gamma_hbm, gamma_v)
        pltpu.sync_copy(beta_hbm, beta_v)
        riota = lax.iota(jnp.int32, 16)

        def chunk_body(g, carry):
            base = wbase + g * _T
            row = wid * n_chunks + g
            pltpu.sync_copy(ids_hbm.at[row], ids_v)

            for j in range(_T // 16):
                sl = pl.ds(j * 16, 16)
                dom = ids_v[pl.ds(0 * _T + j * 16, 16)]
                snp = ids_v[pl.ds(1 * _T + j * 16, 16)]
                pt = ids_v[pl.ds(2 * _T + j * 16, 16)]
                pv = ids_v[pl.ds(3 * _T + j * 16, 16)]
                pos = ids_v[pl.ds(4 * _T + j * 16, 16)]
                cidx_v[sl] = dom * 1600 + snp * 100 + pt
                pv_v[sl] = pv
                mpos_v[sl] = jnp.where(dom == SNP_DOMAIN, pos,
                                       n_pos + (pos & (_N_PAD - 1)))

            c1 = pltpu.async_copy(w_merged_hbm.at[cidx_v], rows_m, sem)
            c1.wait()
            c2 = pltpu.async_copy(w_pv_hbm.at[pv_v], rows_m, sem, add=True)
            c3 = pltpu.async_copy(w_pos_hbm.at[mpos_v], rows_m, sem, add=True)
            c2.wait()
            c3.wait()

            @plsc.parallel_loop(0, _T // 16, step=1)
            def grp_body(q):
                t0 = q * 16
                for i in range(16):
                    t = t0 + i
                    sv = rows_m[t, pl.ds(0, 16)]
                    qv = sv * sv
                    for k in range(1, D // 16):
                        v = rows_m[t, pl.ds(k * 16, 16)]
                        sv = sv + v
                        qv = qv + v * v
                    tb_s[q, i, pl.ds(0, 16)] = sv
                    tb_q[q, i, pl.ds(0, 16)] = qv
                qv16 = jnp.broadcast_to(q, (16,))
                c0 = jnp.full((16,), 0, jnp.int32)
                tot = plsc.load_gather(tb_s, [qv16, riota, c0])
                tot2 = plsc.load_gather(tb_q, [qv16, riota, c0])
                for j in range(1, 16):
                    cj = jnp.full((16,), j, jnp.int32)
                    tot = tot + plsc.load_gather(tb_s, [qv16, riota, cj])
                    tot2 = tot2 + plsc.load_gather(tb_q, [qv16, riota, cj])
                mean = tot * (1.0 / D)
                var = tot2 * (1.0 / D) - mean * mean
                rstd = _rsqrt_newton(var + _LN_EPS)
                nbias = -mean * rstd
                for i in range(16):
                    t = t0 + i
                    sc_i = jnp.broadcast_to(rstd[i], (16,))
                    nb_i = jnp.broadcast_to(nbias[i], (16,))
                    for k in range(D // 16):
                        sl = pl.ds(k * 16, 16)
                        v = rows_m[t, sl]
                        rows_o[t, sl] = ((v * sc_i + nb_i) * gamma_v[sl]
                                         + beta_v[sl])
            pltpu.sync_copy(rows_o, out_hbm.at[pl.ds(base, _T)])
            return carry

        lax.fori_loop(0, n_chunks, chunk_body, 0)

    return body


def kernel(domain_ids, snp_value_ids, snp_position_ids, phenotype_value_ids,
           phenotype_type_ids, is_padding, W_domain, W_snp, W_phen_val,
           W_phen_type, W_pos, ln_gamma, ln_beta):
    B, L = domain_ids.shape
    BL = B * L
    n_pos = W_pos.shape[0]
    per_worker = BL // _NW
    n_chunks = per_worker // _T
    w_merged = (W_domain[:, None, None, :] + W_snp[None, :, None, :]
                + W_phen_type[None, None, :, :]).reshape(-1, D)
    w_pos_ext = jnp.concatenate(
        [W_pos, jnp.zeros((_N_PAD, D), W_pos.dtype)], axis=0)
    ids = jnp.stack([
        domain_ids.reshape(-1), snp_value_ids.reshape(-1),
        phenotype_type_ids.reshape(-1), phenotype_value_ids.reshape(-1),
        snp_position_ids.reshape(-1)
    ]).astype(jnp.int32)
    ids = ids.reshape(5, _NW, n_chunks, _T).transpose(1, 2, 0, 3)
    ids = ids.reshape(_NW * n_chunks, 5 * _T)
    sc = _make_sc_kernel(BL, n_pos)
    out = sc(ids, w_merged, W_phen_val, w_pos_ext, ln_gamma, ln_beta)
    return out.reshape(B, L, D)


# speedup vs baseline: 4.6187x; 1.8564x over previous
import functools

import jax
import jax.numpy as jnp
from jax import lax
from jax.experimental import pallas as pl
from jax.experimental.pallas import tpu as pltpu
from jax.experimental.pallas import tpu_sc as plsc

D = 128
SNP_DOMAIN = 2
_NC = 2
_NS = 16
_NW = _NC * _NS
_T = 256
_LN_EPS = 1e-12
_N_PAD = 1024
_RB = 2048


def _make_sc_kernel(BL, n_pos):
    per_worker = BL // _NW
    n_chunks = per_worker // _T
    mesh = plsc.VectorSubcoreMesh(core_axis_name="c", subcore_axis_name="s")

    @functools.partial(
        pl.kernel,
        mesh=mesh,
        compiler_params=pltpu.CompilerParams(needs_layout_passes=False),
        out_type=jax.ShapeDtypeStruct((BL, D), jnp.float32),
        scratch_types=[
            pltpu.VMEM((5 * _T,), jnp.int32),
            pltpu.VMEM((_T,), jnp.int32),
            pltpu.VMEM((_T,), jnp.int32),
            pltpu.VMEM((_T,), jnp.int32),
            pltpu.VMEM((_T, D), jnp.float32),
            pltpu.SemaphoreType.DMA,
        ],
    )
    def body(ids_hbm, w_merged_hbm, w_pv_hbm, w_pos_hbm, out_hbm,
             ids_v, cidx_v, pv_v, mpos_v, rows_m, sem):
        wid = lax.axis_index("s") * _NC + lax.axis_index("c")
        wbase = wid * per_worker

        def chunk_body(g, carry):
            base = wbase + g * _T
            row = wid * n_chunks + g
            pltpu.sync_copy(ids_hbm.at[row], ids_v)

            for j in range(_T // 16):
                sl = pl.ds(j * 16, 16)
                dom = ids_v[pl.ds(0 * _T + j * 16, 16)]
                snp = ids_v[pl.ds(1 * _T + j * 16, 16)]
                pt = ids_v[pl.ds(2 * _T + j * 16, 16)]
                pv = ids_v[pl.ds(3 * _T + j * 16, 16)]
                pos = ids_v[pl.ds(4 * _T + j * 16, 16)]
                cidx_v[sl] = dom * 1600 + snp * 100 + pt
                pv_v[sl] = pv
                mpos_v[sl] = jnp.where(dom == SNP_DOMAIN, pos,
                                       n_pos + (pos & (_N_PAD - 1)))

            c1 = pltpu.async_copy(w_merged_hbm.at[cidx_v], rows_m, sem)
            c1.wait()
            c2 = pltpu.async_copy(w_pv_hbm.at[pv_v], rows_m, sem, add=True)
            c3 = pltpu.async_copy(w_pos_hbm.at[mpos_v], rows_m, sem, add=True)
            c2.wait()
            c3.wait()
            pltpu.sync_copy(rows_m, out_hbm.at[pl.ds(base, _T)])
            return carry

        lax.fori_loop(0, n_chunks, chunk_body, 0)

    return body


def _ln_block(x_ref, g_ref, b_ref, o_ref):
    x = x_ref[...]
    mean = jnp.mean(x, axis=1, keepdims=True)
    var = jnp.mean(x * x, axis=1, keepdims=True) - mean * mean
    rstd = lax.rsqrt(var + _LN_EPS)
    o_ref[...] = (x - mean) * rstd * g_ref[...] + b_ref[...]


def _ln_tc(summed, ln_gamma, ln_beta):
    BL = summed.shape[0]
    grid = (BL // _RB,)
    return pl.pallas_call(
        _ln_block,
        grid=grid,
        in_specs=[
            pl.BlockSpec((_RB, D), lambda i: (i, 0)),
            pl.BlockSpec((1, D), lambda i: (0, 0)),
            pl.BlockSpec((1, D), lambda i: (0, 0)),
        ],
        out_specs=pl.BlockSpec((_RB, D), lambda i: (i, 0)),
        out_shape=jax.ShapeDtypeStruct((BL, D), jnp.float32),
    )(summed, ln_gamma.reshape(1, D), ln_beta.reshape(1, D))


def kernel(domain_ids, snp_value_ids, snp_position_ids, phenotype_value_ids,
           phenotype_type_ids, is_padding, W_domain, W_snp, W_phen_val,
           W_phen_type, W_pos, ln_gamma, ln_beta):
    B, L = domain_ids.shape
    BL = B * L
    n_pos = W_pos.shape[0]
    per_worker = BL // _NW
    n_chunks = per_worker // _T
    w_merged = (W_domain[:, None, None, :] + W_snp[None, :, None, :]
                + W_phen_type[None, None, :, :]).reshape(-1, D)
    w_pos_ext = jnp.concatenate(
        [W_pos, jnp.zeros((_N_PAD, D), W_pos.dtype)], axis=0)
    ids = jnp.stack([
        domain_ids.reshape(-1), snp_value_ids.reshape(-1),
        phenotype_type_ids.reshape(-1), phenotype_value_ids.reshape(-1),
        snp_position_ids.reshape(-1)
    ]).astype(jnp.int32)
    ids = ids.reshape(5, _NW, n_chunks, _T).transpose(1, 2, 0, 3)
    ids = ids.reshape(_NW * n_chunks, 5 * _T)

    sc = _make_sc_kernel(BL, n_pos)
    summed = sc(ids, w_merged, W_phen_val, w_pos_ext)
    out = _ln_tc(summed, ln_gamma, ln_beta)
    return out.reshape(B, L, D)


# speedup vs baseline: 5.0189x; 1.0866x over previous
import functools

import jax
import jax.numpy as jnp
from jax import lax
from jax.experimental import pallas as pl
from jax.experimental.pallas import tpu as pltpu
from jax.experimental.pallas import tpu_sc as plsc

D = 128
SNP_DOMAIN = 2
_NC = 2
_NS = 16
_NW = _NC * _NS
_T = 256
_LN_EPS = 1e-12
_N_PAD = 1024
_RB = 2048


def _make_sc_kernel(BL, n_pos):
    per_worker = BL // _NW
    n_chunks = per_worker // _T
    mesh = plsc.VectorSubcoreMesh(core_axis_name="c", subcore_axis_name="s")

    @functools.partial(
        pl.kernel,
        mesh=mesh,
        compiler_params=pltpu.CompilerParams(needs_layout_passes=False),
        out_type=jax.ShapeDtypeStruct((BL, D), jnp.float32),
        scratch_types=[
            [pltpu.VMEM((5 * _T,), jnp.int32)] * 2,
            [pltpu.VMEM((_T,), jnp.int32)] * 2,
            [pltpu.VMEM((_T,), jnp.int32)] * 2,
            [pltpu.VMEM((_T,), jnp.int32)] * 2,
            [pltpu.VMEM((_T, D), jnp.float32)] * 2,
            [pltpu.SemaphoreType.DMA] * 2,
            [pltpu.SemaphoreType.DMA] * 2,
        ],
    )
    def body(ids_hbm, w_merged_hbm, w_pv_hbm, w_pos_hbm, out_hbm,
             ids_v, cidx_v, pv_v, mpos_v, rows_m, sem_g, sem_o):
        wid = lax.axis_index("s") * _NC + lax.axis_index("c")
        wbase = wid * per_worker

        def load_ids(g, p):
            pltpu.sync_copy(ids_hbm.at[wid * n_chunks + g], ids_v[p])
            for j in range(_T // 16):
                sl = pl.ds(j * 16, 16)
                dom = ids_v[p][pl.ds(0 * _T + j * 16, 16)]
                snp = ids_v[p][pl.ds(1 * _T + j * 16, 16)]
                pt = ids_v[p][pl.ds(2 * _T + j * 16, 16)]
                pv = ids_v[p][pl.ds(3 * _T + j * 16, 16)]
                pos = ids_v[p][pl.ds(4 * _T + j * 16, 16)]
                cidx_v[p][sl] = dom * 1600 + snp * 100 + pt
                pv_v[p][sl] = pv
                mpos_v[p][sl] = jnp.where(dom == SNP_DOMAIN, pos,
                                          n_pos + (pos & (_N_PAD - 1)))

        def fire_c1(p):
            pltpu.make_async_copy(w_merged_hbm.at[cidx_v[p]], rows_m[p],
                                  sem_g[p]).start()

        def out_slice(g):
            return out_hbm.at[pl.ds(wbase + g * _T, _T)]

        def step(g, p, pn):
            pltpu.make_async_copy(w_merged_hbm.at[cidx_v[p]], rows_m[p],
                                  sem_g[p]).wait()
            pltpu.make_async_copy(w_pv_hbm.at[pv_v[p]], rows_m[p],
                                  sem_g[p]).start(add=True)
            pltpu.make_async_copy(w_pos_hbm.at[mpos_v[p]], rows_m[p],
                                  sem_g[p]).start(add=True)
            @pl.when(g + 1 < n_chunks)
            def _():
                load_ids(g + 1, pn)

            @pl.when(g >= 1)
            def _():
                pltpu.make_async_copy(rows_m[pn], out_slice(g - 1),
                                      sem_o[pn]).wait()

            @pl.when(g + 1 < n_chunks)
            def _():
                fire_c1(pn)
            pltpu.make_async_copy(w_pv_hbm.at[pv_v[p]], rows_m[p],
                                  sem_g[p]).wait()
            pltpu.make_async_copy(w_pos_hbm.at[mpos_v[p]], rows_m[p],
                                  sem_g[p]).wait()
            pltpu.make_async_copy(rows_m[p], out_slice(g), sem_o[p]).start()

        load_ids(0, 0)
        fire_c1(0)

        def pair_body(i, carry):
            step(2 * i, 0, 1)
            step(2 * i + 1, 1, 0)
            return carry

        lax.fori_loop(0, n_chunks // 2, pair_body, 0)
        pltpu.make_async_copy(rows_m[1], out_slice(n_chunks - 1),
                              sem_o[1]).wait()

    return body


def _ln_block(x_ref, g_ref, b_ref, o_ref):
    x = x_ref[...].astype(jnp.float32)
    mean = jnp.mean(x, axis=1, keepdims=True)
    var = jnp.mean(x * x, axis=1, keepdims=True) - mean * mean
    rstd = lax.rsqrt(var + _LN_EPS)
    o_ref[...] = (x - mean) * rstd * g_ref[...] + b_ref[...]


def _ln_tc(summed, ln_gamma, ln_beta):
    BL = summed.shape[0]
    grid = (BL // _RB,)
    return pl.pallas_call(
        _ln_block,
        grid=grid,
        in_specs=[
            pl.BlockSpec((_RB, D), lambda i: (i, 0)),
            pl.BlockSpec((1, D), lambda i: (0, 0)),
            pl.BlockSpec((1, D), lambda i: (0, 0)),
        ],
        out_specs=pl.BlockSpec((_RB, D), lambda i: (i, 0)),
        out_shape=jax.ShapeDtypeStruct((BL, D), jnp.float32),
    )(summed, ln_gamma.reshape(1, D), ln_beta.reshape(1, D))


def kernel(domain_ids, snp_value_ids, snp_position_ids, phenotype_value_ids,
           phenotype_type_ids, is_padding, W_domain, W_snp, W_phen_val,
           W_phen_type, W_pos, ln_gamma, ln_beta):
    B, L = domain_ids.shape
    BL = B * L
    n_pos = W_pos.shape[0]
    per_worker = BL // _NW
    n_chunks = per_worker // _T
    w_merged = (W_domain[:, None, None, :] + W_snp[None, :, None, :]
                + W_phen_type[None, None, :, :]).reshape(-1, D)
    w_pos_ext = jnp.concatenate(
        [W_pos, jnp.zeros((_N_PAD, D), W_pos.dtype)], axis=0)
    ids = jnp.stack([
        domain_ids.reshape(-1), snp_value_ids.reshape(-1),
        phenotype_type_ids.reshape(-1), phenotype_value_ids.reshape(-1),
        snp_position_ids.reshape(-1)
    ]).astype(jnp.int32)
    ids = ids.reshape(5, _NW, n_chunks, _T).transpose(1, 2, 0, 3)
    ids = ids.reshape(_NW * n_chunks, 5 * _T)

    sc = _make_sc_kernel(BL, n_pos)
    summed = sc(ids, w_merged, W_phen_val, w_pos_ext)
    out = _ln_tc(summed, ln_gamma, ln_beta)
    return out.reshape(B, L, D)


# speedup vs baseline: 5.6700x; 1.1297x over previous
import functools

import jax
import jax.numpy as jnp
from jax import lax
from jax.experimental import pallas as pl
from jax.experimental.pallas import tpu as pltpu
from jax.experimental.pallas import tpu_sc as plsc

D = 128
SNP_DOMAIN = 2
_NC = 2
_NS = 16
_NW = _NC * _NS
_T = 256
_LN_EPS = 1e-12
_N_PAD = 1024
_RB = 8192


def _make_sc_kernel(BL, n_pos):
    per_worker = BL // _NW
    n_chunks = per_worker // _T
    mesh = plsc.VectorSubcoreMesh(core_axis_name="c", subcore_axis_name="s")

    @functools.partial(
        pl.kernel,
        mesh=mesh,
        compiler_params=pltpu.CompilerParams(needs_layout_passes=False),
        out_type=jax.ShapeDtypeStruct((BL, D), jnp.float32),
        scratch_types=[
            [pltpu.VMEM((5 * _T,), jnp.int32)] * 2,
            [pltpu.VMEM((_T,), jnp.int32)] * 2,
            [pltpu.VMEM((_T,), jnp.int32)] * 2,
            [pltpu.VMEM((_T,), jnp.int32)] * 2,
            [pltpu.VMEM((_T, D), jnp.float32)] * 2,
            [pltpu.SemaphoreType.DMA] * 2,
            [pltpu.SemaphoreType.DMA] * 2,
        ],
    )
    def body(ids_hbm, w_merged_hbm, w_pv_hbm, w_pos_hbm, out_hbm,
             ids_v, cidx_v, pv_v, mpos_v, rows_m, sem_g, sem_o):
        wid = lax.axis_index("s") * _NC + lax.axis_index("c")
        wbase = wid * per_worker

        def load_ids(g, p):
            pltpu.sync_copy(ids_hbm.at[wid * n_chunks + g], ids_v[p])
            for j in range(_T // 16):
                sl = pl.ds(j * 16, 16)
                dom = ids_v[p][pl.ds(0 * _T + j * 16, 16)]
                snp = ids_v[p][pl.ds(1 * _T + j * 16, 16)]
                pt = ids_v[p][pl.ds(2 * _T + j * 16, 16)]
                pv = ids_v[p][pl.ds(3 * _T + j * 16, 16)]
                pos = ids_v[p][pl.ds(4 * _T + j * 16, 16)]
                cidx_v[p][sl] = dom * 1600 + snp * 100 + pt
                pv_v[p][sl] = pv
                mpos_v[p][sl] = jnp.where(dom == SNP_DOMAIN, pos,
                                          n_pos + (pos & (_N_PAD - 1)))

        def fire_c1(p):
            pltpu.make_async_copy(w_merged_hbm.at[cidx_v[p]], rows_m[p],
                                  sem_g[p]).start()

        def out_slice(g):
            return out_hbm.at[pl.ds(wbase + g * _T, _T)]

        def step(g, p, pn):
            pltpu.make_async_copy(w_merged_hbm.at[cidx_v[p]], rows_m[p],
                                  sem_g[p]).wait()
            pltpu.make_async_copy(w_pv_hbm.at[pv_v[p]], rows_m[p],
                                  sem_g[p]).start(add=True)
            pltpu.make_async_copy(w_pos_hbm.at[mpos_v[p]], rows_m[p],
                                  sem_g[p]).start(add=True)
            @pl.when(g + 1 < n_chunks)
            def _():
                load_ids(g + 1, pn)

            @pl.when(g >= 1)
            def _():
                pltpu.make_async_copy(rows_m[pn], out_slice(g - 1),
                                      sem_o[pn]).wait()

            @pl.when(g + 1 < n_chunks)
            def _():
                fire_c1(pn)
            pltpu.make_async_copy(w_pv_hbm.at[pv_v[p]], rows_m[p],
                                  sem_g[p]).wait()
            pltpu.make_async_copy(w_pos_hbm.at[mpos_v[p]], rows_m[p],
                                  sem_g[p]).wait()
            pltpu.make_async_copy(rows_m[p], out_slice(g), sem_o[p]).start()

        load_ids(0, 0)
        fire_c1(0)

        def pair_body(i, carry):
            step(2 * i, 0, 1)
            step(2 * i + 1, 1, 0)
            return carry

        lax.fori_loop(0, n_chunks // 2, pair_body, 0)
        pltpu.make_async_copy(rows_m[1], out_slice(n_chunks - 1),
                              sem_o[1]).wait()

    return body


def _ln_block(x_ref, g_ref, b_ref, o_ref):
    x = x_ref[...].astype(jnp.float32)
    mean = jnp.mean(x, axis=1, keepdims=True)
    var = jnp.mean(x * x, axis=1, keepdims=True) - mean * mean
    rstd = lax.rsqrt(var + _LN_EPS)
    o_ref[...] = (x - mean) * rstd * g_ref[...] + b_ref[...]


def _ln_tc(summed, ln_gamma, ln_beta):
    BL = summed.shape[0]
    grid = (BL // _RB,)
    return pl.pallas_call(
        _ln_block,
        grid=grid,
        in_specs=[
            pl.BlockSpec((_RB, D), lambda i: (i, 0)),
            pl.BlockSpec((1, D), lambda i: (0, 0)),
            pl.BlockSpec((1, D), lambda i: (0, 0)),
        ],
        out_specs=pl.BlockSpec((_RB, D), lambda i: (i, 0)),
        out_shape=jax.ShapeDtypeStruct((BL, D), jnp.float32),
    )(summed, ln_gamma.reshape(1, D), ln_beta.reshape(1, D))


def kernel(domain_ids, snp_value_ids, snp_position_ids, phenotype_value_ids,
           phenotype_type_ids, is_padding, W_domain, W_snp, W_phen_val,
           W_phen_type, W_pos, ln_gamma, ln_beta):
    B, L = domain_ids.shape
    BL = B * L
    n_pos = W_pos.shape[0]
    per_worker = BL // _NW
    n_chunks = per_worker // _T
    w_merged = (W_domain[:, None, None, :] + W_snp[None, :, None, :]
                + W_phen_type[None, None, :, :]).reshape(-1, D)
    w_pos_ext = jnp.concatenate(
        [W_pos, jnp.zeros((_N_PAD, D), W_pos.dtype)], axis=0)
    ids = jnp.stack([
        domain_ids.reshape(-1), snp_value_ids.reshape(-1),
        phenotype_type_ids.reshape(-1), phenotype_value_ids.reshape(-1),
        snp_position_ids.reshape(-1)
    ]).astype(jnp.int32)
    ids = ids.reshape(5, _NW, n_chunks, _T).transpose(1, 2, 0, 3)
    ids = ids.reshape(_NW * n_chunks, 5 * _T)

    sc = _make_sc_kernel(BL, n_pos)
    summed = sc(ids, w_merged, W_phen_val, w_pos_ext)
    out = _ln_tc(summed, ln_gamma, ln_beta)
    return out.reshape(B, L, D)


# speedup vs baseline: 5.8161x; 1.0258x over previous
import functools

import jax
import jax.numpy as jnp
from jax import lax
from jax.experimental import pallas as pl
from jax.experimental.pallas import tpu as pltpu
from jax.experimental.pallas import tpu_sc as plsc

D = 128
SNP_DOMAIN = 2
_NC = 2
_NS = 16
_NW = _NC * _NS
_T = 256
_LN_EPS = 1e-12
_N_PAD = 1024
_RB = 16384


def _make_sc_kernel(BL, n_pos):
    per_worker = BL // _NW
    n_chunks = per_worker // _T
    mesh = plsc.VectorSubcoreMesh(core_axis_name="c", subcore_axis_name="s")

    @functools.partial(
        pl.kernel,
        mesh=mesh,
        compiler_params=pltpu.CompilerParams(needs_layout_passes=False),
        out_type=jax.ShapeDtypeStruct((BL, D), jnp.float32),
        scratch_types=[
            [pltpu.VMEM((5 * _T,), jnp.int32)] * 2,
            [pltpu.VMEM((_T,), jnp.int32)] * 2,
            [pltpu.VMEM((_T,), jnp.int32)] * 2,
            [pltpu.VMEM((_T,), jnp.int32)] * 2,
            [pltpu.VMEM((_T, D), jnp.float32)] * 2,
            [pltpu.SemaphoreType.DMA] * 2,
            [pltpu.SemaphoreType.DMA] * 2,
        ],
    )
    def body(ids_hbm, w_merged_hbm, w_pv_hbm, w_pos_hbm, out_hbm,
             ids_v, cidx_v, pv_v, mpos_v, rows_m, sem_g, sem_o):
        wid = lax.axis_index("s") * _NC + lax.axis_index("c")
        wbase = wid * per_worker

        def load_ids(g, p):
            pltpu.sync_copy(ids_hbm.at[wid * n_chunks + g], ids_v[p])
            for j in range(_T // 16):
                sl = pl.ds(j * 16, 16)
                dom = ids_v[p][pl.ds(0 * _T + j * 16, 16)]
                snp = ids_v[p][pl.ds(1 * _T + j * 16, 16)]
                pt = ids_v[p][pl.ds(2 * _T + j * 16, 16)]
                pv = ids_v[p][pl.ds(3 * _T + j * 16, 16)]
                pos = ids_v[p][pl.ds(4 * _T + j * 16, 16)]
                cidx_v[p][sl] = dom * 1600 + snp * 100 + pt
                pv_v[p][sl] = pv
                mpos_v[p][sl] = jnp.where(dom == SNP_DOMAIN, pos,
                                          n_pos + (pos & (_N_PAD - 1)))

        def fire_c1(p):
            pltpu.make_async_copy(w_merged_hbm.at[cidx_v[p]], rows_m[p],
                                  sem_g[p]).start()

        def out_slice(g):
            return out_hbm.at[pl.ds(wbase + g * _T, _T)]

        def step(g, p, pn):
            pltpu.make_async_copy(w_merged_hbm.at[cidx_v[p]], rows_m[p],
                                  sem_g[p]).wait()
            pltpu.make_async_copy(w_pv_hbm.at[pv_v[p]], rows_m[p],
                                  sem_g[p]).start(add=True)
            pltpu.make_async_copy(w_pos_hbm.at[mpos_v[p]], rows_m[p],
                                  sem_g[p]).start(add=True)
            @pl.when(g + 1 < n_chunks)
            def _():
                load_ids(g + 1, pn)

            @pl.when(g >= 1)
            def _():
                pltpu.make_async_copy(rows_m[pn], out_slice(g - 1),
                                      sem_o[pn]).wait()

            @pl.when(g + 1 < n_chunks)
            def _():
                fire_c1(pn)
            pltpu.make_async_copy(w_pv_hbm.at[pv_v[p]], rows_m[p],
                                  sem_g[p]).wait()
            pltpu.make_async_copy(w_pos_hbm.at[mpos_v[p]], rows_m[p],
                                  sem_g[p]).wait()
            pltpu.make_async_copy(rows_m[p], out_slice(g), sem_o[p]).start()

        load_ids(0, 0)
        fire_c1(0)

        def pair_body(i, carry):
            step(2 * i, 0, 1)
            step(2 * i + 1, 1, 0)
            return carry

        lax.fori_loop(0, n_chunks // 2, pair_body, 0)
        pltpu.make_async_copy(rows_m[1], out_slice(n_chunks - 1),
                              sem_o[1]).wait()

    return body


def _ln_block(x_ref, g_ref, b_ref, o_ref):
    x = x_ref[...].astype(jnp.float32)
    mean = jnp.mean(x, axis=1, keepdims=True)
    var = jnp.mean(x * x, axis=1, keepdims=True) - mean * mean
    rstd = lax.rsqrt(var + _LN_EPS)
    o_ref[...] = (x - mean) * rstd * g_ref[...] + b_ref[...]


def _ln_tc(summed, ln_gamma, ln_beta):
    BL = summed.shape[0]
    grid = (BL // _RB,)
    return pl.pallas_call(
        _ln_block,
        grid=grid,
        in_specs=[
            pl.BlockSpec((_RB, D), lambda i: (i, 0)),
            pl.BlockSpec((1, D), lambda i: (0, 0)),
            pl.BlockSpec((1, D), lambda i: (0, 0)),
        ],
        out_specs=pl.BlockSpec((_RB, D), lambda i: (i, 0)),
        out_shape=jax.ShapeDtypeStruct((BL, D), jnp.float32),
    )(summed, ln_gamma.reshape(1, D), ln_beta.reshape(1, D))


def kernel(domain_ids, snp_value_ids, snp_position_ids, phenotype_value_ids,
           phenotype_type_ids, is_padding, W_domain, W_snp, W_phen_val,
           W_phen_type, W_pos, ln_gamma, ln_beta):
    B, L = domain_ids.shape
    BL = B * L
    n_pos = W_pos.shape[0]
    per_worker = BL // _NW
    n_chunks = per_worker // _T
    w_merged = (W_domain[:, None, None, :] + W_snp[None, :, None, :]
                + W_phen_type[None, None, :, :]).reshape(-1, D)
    w_pos_ext = jnp.concatenate(
        [W_pos, jnp.zeros((_N_PAD, D), W_pos.dtype)], axis=0)
    ids = jnp.stack([
        domain_ids.reshape(-1), snp_value_ids.reshape(-1),
        phenotype_type_ids.reshape(-1), phenotype_value_ids.reshape(-1),
        snp_position_ids.reshape(-1)
    ]).astype(jnp.int32)
    ids = ids.reshape(5, _NW, n_chunks, _T).transpose(1, 2, 0, 3)
    ids = ids.reshape(_NW * n_chunks, 5 * _T)

    sc = _make_sc_kernel(BL, n_pos)
    summed = sc(ids, w_merged, W_phen_val, w_pos_ext)
    out = _ln_tc(summed, ln_gamma, ln_beta)
    return out.reshape(B, L, D)
